# baseline (speedup 1.0000x reference)
import jax
import jax.numpy as jnp
from jax import lax
from jax.experimental import pallas as pl
from jax.experimental.pallas import tpu as pltpu
from jax.experimental.pallas import tpu_sc as plsc

VOCAB_ROWS = 1_000_000
EMB_DIM = 32
NUM_TERMS = 50
NUM_BAGS = 16384
NUM_CORES = 2
NUM_SUBCORES = 16
NUM_WORKERS = NUM_CORES * NUM_SUBCORES
BAGS_PER_WORKER = NUM_BAGS // NUM_WORKERS
CHUNK = 128
CHUNKS_PER_TERM = BAGS_PER_WORKER // CHUNK


NBUF = 8
NUM_CHUNKS = NUM_TERMS * CHUNKS_PER_TERM


BATCH = 16


def _accumulate(acc, buf, c):
    for r0 in range(0, CHUNK, BATCH):
        xs = [buf[r, pl.ds(h, 16)]
              for r in range(r0, r0 + BATCH) for h in (0, 16)]
        for i, r in enumerate(range(r0, r0 + BATCH)):
            b = c * CHUNK + r
            plsc.addupdate(acc.at[b, pl.ds(0, 16)], xs[2 * i])
            plsc.addupdate(acc.at[b, pl.ds(16, 16)], xs[2 * i + 1])


def _bag_sum_body(idx_hbm, weight_hbm, out_hbm, idx_v, acc, *ring):
    bufs = ring[:NBUF]
    sems = ring[NBUF:]
    wtab = weight_hbm
    wid = lax.axis_index("s") * NUM_CORES + lax.axis_index("c")
    base = wid * BAGS_PER_WORKER

    pltpu.sync_copy(idx_hbm.at[:, pl.ds(wid * CHUNKS_PER_TERM,
                                        CHUNKS_PER_TERM)], idx_v)

    zero = jnp.zeros((16,), jnp.float32)

    @pl.loop(0, BAGS_PER_WORKER)
    def _(i):
        acc[i, pl.ds(0, 16)] = zero
        acc[i, pl.ds(16, 16)] = zero

    def start(l, c, b):
        pltpu.async_copy(wtab.at[idx_v.at[l, c]], bufs[b], sems[b])

    def wait(b):
        pltpu.make_async_copy(wtab.at[pl.ds(0, CHUNK)], bufs[b],
                              sems[b]).wait()

    for b in range(NBUF):
        start(b // CHUNKS_PER_TERM, b % CHUNKS_PER_TERM, b)

    @pl.loop(0, NUM_CHUNKS, step=NBUF)
    def _(j):
        for b in range(NBUF):
            jj = j + b
            wait(b)
            _accumulate(acc, bufs[b], b % CHUNKS_PER_TERM)
            nxt = jj + NBUF

            @pl.when(nxt < NUM_CHUNKS)
            def _():
                start(nxt // CHUNKS_PER_TERM, b % CHUNKS_PER_TERM, b)

    pltpu.sync_copy(acc, out_hbm.at[pl.ds(base, BAGS_PER_WORKER)])


TR_BLOCK = 8192


def _transpose_block(wt_ref, out_ref):
    eye = jnp.eye(EMB_DIM, dtype=jnp.float32)
    out_ref[...] = jax.lax.dot_general(
        wt_ref[...], eye, (((0,), (0,)), ((), ())),
        preferred_element_type=jnp.float32)


def _to_row_major(wt):
    grid = (VOCAB_ROWS + TR_BLOCK - 1) // TR_BLOCK
    return pl.pallas_call(
        _transpose_block,
        grid=(grid,),
        in_specs=[pl.BlockSpec((EMB_DIM, TR_BLOCK), lambda i: (0, i))],
        out_specs=pl.BlockSpec((TR_BLOCK, EMB_DIM), lambda i: (i, 0)),
        out_shape=jax.ShapeDtypeStruct((VOCAB_ROWS, EMB_DIM), jnp.float32),
    )(wt)


def kernel(input, weight):
    idx = input.astype(jnp.int32).reshape(NUM_TERMS, NUM_BAGS // CHUNK, CHUNK)
    mesh = plsc.VectorSubcoreMesh(core_axis_name="c", subcore_axis_name="s")
    run = pl.kernel(
        _bag_sum_body,
        out_type=jax.ShapeDtypeStruct((NUM_BAGS, EMB_DIM), jnp.float32),
        mesh=mesh,
        compiler_params=pltpu.CompilerParams(use_tc_tiling_on_sc=False),
        scratch_types=(
            [pltpu.VMEM((NUM_TERMS, CHUNKS_PER_TERM, CHUNK), jnp.int32),
             pltpu.VMEM((BAGS_PER_WORKER, EMB_DIM), jnp.float32)]
            + [pltpu.VMEM((CHUNK, EMB_DIM), jnp.float32)] * NBUF
            + [pltpu.SemaphoreType.DMA] * NBUF
        ),
    )
    return run(idx, _to_row_major(jnp.swapaxes(weight, 0, 1)))


# speedup vs baseline: 1.9953x; 1.9953x over previous
import jax
import jax.numpy as jnp
from jax import lax
from jax.experimental import pallas as pl
from jax.experimental.pallas import tpu as pltpu
from jax.experimental.pallas import tpu_sc as plsc

VOCAB_ROWS = 1_000_000
EMB_DIM = 32
NUM_TERMS = 50
NUM_BAGS = 16384
NUM_CORES = 2
NUM_SUBCORES = 16
NUM_WORKERS = NUM_CORES * NUM_SUBCORES
BAGS_PER_WORKER = NUM_BAGS // NUM_WORKERS
CHUNK = 128
CHUNKS_PER_TERM = BAGS_PER_WORKER // CHUNK


NBUF = 8
NUM_CHUNKS = NUM_TERMS * CHUNKS_PER_TERM


BATCH = 16


def _accumulate(acc, buf, c):
    for r0 in range(0, CHUNK, BATCH):
        xs = [buf[r, pl.ds(h, 16)]
              for r in range(r0, r0 + BATCH) for h in (0, 16)]
        for i, r in enumerate(range(r0, r0 + BATCH)):
            b = c * CHUNK + r
            plsc.addupdate(acc.at[b, pl.ds(0, 16)], xs[2 * i])
            plsc.addupdate(acc.at[b, pl.ds(16, 16)], xs[2 * i + 1])


def _bag_sum_body(idx_hbm, weight_hbm, out_hbm, idx_v, acc, *ring):
    bufs = ring[:NBUF]
    sems = ring[NBUF:]
    wtab = weight_hbm
    wid = lax.axis_index("s") * NUM_CORES + lax.axis_index("c")
    base = wid * BAGS_PER_WORKER

    pltpu.sync_copy(idx_hbm.at[:, pl.ds(wid * CHUNKS_PER_TERM,
                                        CHUNKS_PER_TERM)], idx_v)

    zero = jnp.zeros((16,), jnp.float32)

    @pl.loop(0, BAGS_PER_WORKER)
    def _(i):
        acc[i, pl.ds(0, 16)] = zero
        acc[i, pl.ds(16, 16)] = zero

    def start(l, c, b):
        pltpu.async_copy(wtab.at[idx_v.at[l, c]], bufs[b], sems[b])

    def wait(b):
        pltpu.make_async_copy(wtab.at[pl.ds(0, CHUNK)], bufs[b],
                              sems[b]).wait()

    for b in range(NBUF):
        start(b // CHUNKS_PER_TERM, b % CHUNKS_PER_TERM, b)

    @pl.loop(0, NUM_CHUNKS, step=NBUF)
    def _(j):
        for b in range(NBUF):
            jj = j + b
            wait(b)
            _accumulate(acc, bufs[b], b % CHUNKS_PER_TERM)
            nxt = jj + NBUF

            @pl.when(nxt < NUM_CHUNKS)
            def _():
                start(nxt // CHUNKS_PER_TERM, b % CHUNKS_PER_TERM, b)

    pltpu.sync_copy(acc, out_hbm.at[pl.ds(base, BAGS_PER_WORKER)])


TR_BLOCK = 2048
SLAB_BLOCKS = 123
SLAB = TR_BLOCK * SLAB_BLOCKS
TAB_ROWS = 4 * SLAB


def _transpose_block(w0, w1, w2, w3, out_ref):
    w4 = jnp.concatenate([w0[...], w1[...], w2[...], w3[...]], axis=0)
    eye = jnp.eye(4 * EMB_DIM, dtype=jnp.float32)
    out_ref[...] = jax.lax.dot_general(
        w4, eye, (((0,), (0,)), ((), ())),
        precision=jax.lax.Precision.HIGHEST,
        preferred_element_type=jnp.float32)


def _to_row_major(wt):
    last_block = VOCAB_ROWS // TR_BLOCK
    specs = [
        pl.BlockSpec((EMB_DIM, TR_BLOCK),
                     lambda i, q=q: (0, jnp.minimum(q * SLAB_BLOCKS + i,
                                                    last_block)))
        for q in range(4)
    ]
    w128 = pl.pallas_call(
        _transpose_block,
        grid=(SLAB_BLOCKS,),
        in_specs=specs,
        out_specs=pl.BlockSpec((TR_BLOCK, 4 * EMB_DIM), lambda i: (i, 0)),
        out_shape=jax.ShapeDtypeStruct((SLAB, 4 * EMB_DIM), jnp.float32),
    )(wt, wt, wt, wt)
    return w128.reshape(TAB_ROWS, EMB_DIM)


def kernel(input, weight):
    idx32 = input.astype(jnp.int32)
    perm = 4 * (idx32 % SLAB) + idx32 // SLAB
    idx = perm.reshape(NUM_TERMS, NUM_BAGS // CHUNK, CHUNK)
    mesh = plsc.VectorSubcoreMesh(core_axis_name="c", subcore_axis_name="s")
    run = pl.kernel(
        _bag_sum_body,
        out_type=jax.ShapeDtypeStruct((NUM_BAGS, EMB_DIM), jnp.float32),
        mesh=mesh,
        compiler_params=pltpu.CompilerParams(use_tc_tiling_on_sc=False),
        scratch_types=(
            [pltpu.VMEM((NUM_TERMS, CHUNKS_PER_TERM, CHUNK), jnp.int32),
             pltpu.VMEM((BAGS_PER_WORKER, EMB_DIM), jnp.float32)]
            + [pltpu.VMEM((CHUNK, EMB_DIM), jnp.float32)] * NBUF
            + [pltpu.SemaphoreType.DMA] * NBUF
        ),
    )
    return run(idx, _to_row_major(jnp.swapaxes(weight, 0, 1)))


# speedup vs baseline: 2.2024x; 1.1038x over previous
import jax
import jax.numpy as jnp
from jax import lax
from jax.experimental import pallas as pl
from jax.experimental.pallas import tpu as pltpu
from jax.experimental.pallas import tpu_sc as plsc

VOCAB_ROWS = 1_000_000
EMB_DIM = 32
NUM_TERMS = 50
NUM_BAGS = 16384
NUM_CORES = 2
NUM_SUBCORES = 16
NUM_WORKERS = NUM_CORES * NUM_SUBCORES
BAGS_PER_WORKER = NUM_BAGS // NUM_WORKERS
CHUNK = 128
CHUNKS_PER_TERM = BAGS_PER_WORKER // CHUNK


NBUF = 8
NUM_CHUNKS = NUM_TERMS * CHUNKS_PER_TERM


BATCH = 16


GROUP = 8


def _accumulate(acc, buf, c):
    ngroups = CHUNK // GROUP
    cur = [buf[r, pl.ds(h, 16)] for r in range(GROUP) for h in (0, 16)]
    for k in range(ngroups):
        nxt = []
        for i in range(2 * GROUP):
            if k + 1 < ngroups:
                r = (k + 1) * GROUP + i // 2
                nxt.append(buf[r, pl.ds(16 * (i % 2), 16)])
            r0 = k * GROUP + i // 2
            plsc.addupdate(acc.at[c * CHUNK + r0, pl.ds(16 * (i % 2), 16)],
                           cur[i])
        cur = nxt


def _bag_sum_body(idx_hbm, weight_hbm, out_hbm, idx_v, acc, *ring):
    bufs = ring[:NBUF]
    sems = ring[NBUF:]
    wtab = weight_hbm
    wid = lax.axis_index("s") * NUM_CORES + lax.axis_index("c")
    base = wid * BAGS_PER_WORKER

    pltpu.sync_copy(idx_hbm.at[:, pl.ds(wid * CHUNKS_PER_TERM,
                                        CHUNKS_PER_TERM)], idx_v)

    zero = jnp.zeros((16,), jnp.float32)

    @pl.loop(0, BAGS_PER_WORKER)
    def _(i):
        acc[i, pl.ds(0, 16)] = zero
        acc[i, pl.ds(16, 16)] = zero

    def start(l, c, b):
        pltpu.async_copy(wtab.at[idx_v.at[l, c]], bufs[b], sems[b])

    def wait(b):
        pltpu.make_async_copy(wtab.at[pl.ds(0, CHUNK)], bufs[b],
                              sems[b]).wait()

    for b in range(NBUF):
        start(b // CHUNKS_PER_TERM, b % CHUNKS_PER_TERM, b)

    @pl.loop(0, NUM_CHUNKS, step=NBUF)
    def _(j):
        for b in range(NBUF):
            jj = j + b
            wait(b)
            _accumulate(acc, bufs[b], b % CHUNKS_PER_TERM)
            nxt = jj + NBUF

            @pl.when(nxt < NUM_CHUNKS)
            def _():
                start(nxt // CHUNKS_PER_TERM, b % CHUNKS_PER_TERM, b)

    pltpu.sync_copy(acc, out_hbm.at[pl.ds(base, BAGS_PER_WORKER)])


TR_BLOCK = 4096
SLAB_BLOCKS = 62
SLAB = TR_BLOCK * SLAB_BLOCKS
TAB_ROWS = 4 * SLAB


def _transpose_block(w0, w1, w2, w3, out_ref):
    w4 = jnp.concatenate([w0[...], w1[...], w2[...], w3[...]], axis=0)
    eye = jnp.eye(4 * EMB_DIM, dtype=jnp.float32)
    out_ref[...] = jax.lax.dot_general(
        w4, eye, (((0,), (0,)), ((), ())),
        precision=jax.lax.Precision.HIGHEST,
        preferred_element_type=jnp.float32)


def _to_row_major(wt):
    last_block = VOCAB_ROWS // TR_BLOCK
    specs = [
        pl.BlockSpec((EMB_DIM, TR_BLOCK),
                     lambda i, q=q: (0, jnp.minimum(q * SLAB_BLOCKS + i,
                                                    last_block)))
        for q in range(4)
    ]
    w128 = pl.pallas_call(
        _transpose_block,
        grid=(SLAB_BLOCKS,),
        in_specs=specs,
        out_specs=pl.BlockSpec((TR_BLOCK, 4 * EMB_DIM), lambda i: (i, 0)),
        out_shape=jax.ShapeDtypeStruct((SLAB, 4 * EMB_DIM), jnp.float32),
    )(wt, wt, wt, wt)
    return w128.reshape(TAB_ROWS, EMB_DIM)


def kernel(input, weight):
    idx32 = input.astype(jnp.int32)
    perm = 4 * (idx32 % SLAB) + idx32 // SLAB
    idx = perm.reshape(NUM_TERMS, NUM_BAGS // CHUNK, CHUNK)
    mesh = plsc.VectorSubcoreMesh(core_axis_name="c", subcore_axis_name="s")
    run = pl.kernel(
        _bag_sum_body,
        out_type=jax.ShapeDtypeStruct((NUM_BAGS, EMB_DIM), jnp.float32),
        mesh=mesh,
        compiler_params=pltpu.CompilerParams(use_tc_tiling_on_sc=False),
        scratch_types=(
            [pltpu.VMEM((NUM_TERMS, CHUNKS_PER_TERM, CHUNK), jnp.int32),
             pltpu.VMEM((BAGS_PER_WORKER, EMB_DIM), jnp.float32)]
            + [pltpu.VMEM((CHUNK, EMB_DIM), jnp.float32)] * NBUF
            + [pltpu.SemaphoreType.DMA] * NBUF
        ),
    )
    return run(idx, _to_row_major(jnp.swapaxes(weight, 0, 1)))


# speedup vs baseline: 2.7782x; 1.2614x over previous
import jax
import jax.numpy as jnp
from jax import lax
from jax.experimental import pallas as pl
from jax.experimental.pallas import tpu as pltpu
from jax.experimental.pallas import tpu_sc as plsc

VOCAB_ROWS = 1_000_000
EMB_DIM = 32
NUM_TERMS = 50
NUM_BAGS = 16384
NUM_CORES = 2
NUM_SUBCORES = 16
NUM_WORKERS = NUM_CORES * NUM_SUBCORES
BAGS_PER_WORKER = NUM_BAGS // NUM_WORKERS
CHUNK = 128
CHUNKS_PER_TERM = BAGS_PER_WORKER // CHUNK


NBUF = 8
NUM_CHUNKS = NUM_TERMS * CHUNKS_PER_TERM


BATCH = 16


GROUP = 8

BAGS_PER_CHUNK = 2
ROWS_PER_CHUNK = BAGS_PER_CHUNK * NUM_TERMS
CHUNKS_PER_WORKER = BAGS_PER_WORKER // BAGS_PER_CHUNK


def _accumulate_bags(out_v, buf, jj):
    accs = [buf[bag * NUM_TERMS, pl.ds(h, 16)]
            for bag in range(BAGS_PER_CHUNK) for h in (0, 16)]
    for r in range(1, NUM_TERMS):
        for bag in range(BAGS_PER_CHUNK):
            for hi, h in enumerate((0, 16)):
                k = bag * 2 + hi
                accs[k] = accs[k] + buf[bag * NUM_TERMS + r, pl.ds(h, 16)]
    for bag in range(BAGS_PER_CHUNK):
        for hi, h in enumerate((0, 16)):
            out_v[jj * BAGS_PER_CHUNK + bag, pl.ds(h, 16)] = accs[bag * 2 + hi]


def _bag_sum_body(idx_hbm, weight_hbm, out_hbm, idx_v, out_v, *ring):
    bufs = ring[:NBUF]
    sems = ring[NBUF:]
    wtab = weight_hbm
    wid = lax.axis_index("s") * NUM_CORES + lax.axis_index("c")
    base = wid * BAGS_PER_WORKER

    pltpu.sync_copy(idx_hbm.at[pl.ds(wid * CHUNKS_PER_WORKER,
                                     CHUNKS_PER_WORKER)], idx_v)

    def start(jj, b):
        pltpu.async_copy(wtab.at[idx_v.at[jj]], bufs[b], sems[b])

    def wait(b):
        pltpu.make_async_copy(wtab.at[pl.ds(0, ROWS_PER_CHUNK)], bufs[b],
                              sems[b]).wait()

    for b in range(NBUF):
        start(b, b)

    @pl.loop(0, CHUNKS_PER_WORKER, step=NBUF)
    def _(j):
        for b in range(NBUF):
            jj = j + b
            wait(b)
            _accumulate_bags(out_v, bufs[b], jj)
            nxt = jj + NBUF

            @pl.when(nxt < CHUNKS_PER_WORKER)
            def _():
                start(nxt, b)

    pltpu.sync_copy(out_v, out_hbm.at[pl.ds(base, BAGS_PER_WORKER)])


TR_BLOCK = 4096
SLAB_BLOCKS = 62
SLAB = TR_BLOCK * SLAB_BLOCKS
TAB_ROWS = 4 * SLAB


def _transpose_block(w0, w1, w2, w3, out_ref):
    w4 = jnp.concatenate([w0[...], w1[...], w2[...], w3[...]], axis=0)
    eye = jnp.eye(4 * EMB_DIM, dtype=jnp.float32)
    out_ref[...] = jax.lax.dot_general(
        w4, eye, (((0,), (0,)), ((), ())),
        precision=jax.lax.Precision.HIGHEST,
        preferred_element_type=jnp.float32)


def _to_row_major(wt):
    last_block = VOCAB_ROWS // TR_BLOCK
    specs = [
        pl.BlockSpec((EMB_DIM, TR_BLOCK),
                     lambda i, q=q: (0, jnp.minimum(q * SLAB_BLOCKS + i,
                                                    last_block)))
        for q in range(4)
    ]
    w128 = pl.pallas_call(
        _transpose_block,
        grid=(SLAB_BLOCKS,),
        in_specs=specs,
        out_specs=pl.BlockSpec((TR_BLOCK, 4 * EMB_DIM), lambda i: (i, 0)),
        out_shape=jax.ShapeDtypeStruct((SLAB, 4 * EMB_DIM), jnp.float32),
    )(wt, wt, wt, wt)
    return w128.reshape(TAB_ROWS, EMB_DIM)


def kernel(input, weight):
    idx32 = input.astype(jnp.int32)
    perm = 4 * (idx32 % SLAB) + idx32 // SLAB
    idx = perm.T.reshape(NUM_BAGS // BAGS_PER_CHUNK, ROWS_PER_CHUNK)
    mesh = plsc.VectorSubcoreMesh(core_axis_name="c", subcore_axis_name="s")
    run = pl.kernel(
        _bag_sum_body,
        out_type=jax.ShapeDtypeStruct((NUM_BAGS, EMB_DIM), jnp.float32),
        mesh=mesh,
        compiler_params=pltpu.CompilerParams(use_tc_tiling_on_sc=False),
        scratch_types=(
            [pltpu.VMEM((CHUNKS_PER_WORKER, ROWS_PER_CHUNK), jnp.int32),
             pltpu.VMEM((BAGS_PER_WORKER, EMB_DIM), jnp.float32)]
            + [pltpu.VMEM((ROWS_PER_CHUNK, EMB_DIM), jnp.float32)] * NBUF
            + [pltpu.SemaphoreType.DMA] * NBUF
        ),
    )
    return run(idx, _to_row_major(jnp.swapaxes(weight, 0, 1)))


# speedup vs baseline: 2.9631x; 1.0666x over previous
import jax
import jax.numpy as jnp
from jax import lax
from jax.experimental import pallas as pl
from jax.experimental.pallas import tpu as pltpu
from jax.experimental.pallas import tpu_sc as plsc

VOCAB_ROWS = 1_000_000
EMB_DIM = 32
NUM_TERMS = 50
NUM_BAGS = 16384
NUM_CORES = 2
NUM_SUBCORES = 16
NUM_WORKERS = NUM_CORES * NUM_SUBCORES
BAGS_PER_WORKER = NUM_BAGS // NUM_WORKERS
CHUNK = 128
CHUNKS_PER_TERM = BAGS_PER_WORKER // CHUNK


NBUF = 8
NUM_CHUNKS = NUM_TERMS * CHUNKS_PER_TERM


BATCH = 16


GROUP = 8

BAGS_PER_CHUNK = 2
ROWS_PER_CHUNK = BAGS_PER_CHUNK * NUM_TERMS
CHUNKS_PER_WORKER = BAGS_PER_WORKER // BAGS_PER_CHUNK


def _accumulate_bags(out_v, buf, jj):
    accs = [buf[bag * NUM_TERMS, pl.ds(h, 16)]
            for bag in range(BAGS_PER_CHUNK) for h in (0, 16)]
    for r in range(1, NUM_TERMS):
        for bag in range(BAGS_PER_CHUNK):
            for hi, h in enumerate((0, 16)):
                k = bag * 2 + hi
                accs[k] = accs[k] + buf[bag * NUM_TERMS + r, pl.ds(h, 16)]
    for bag in range(BAGS_PER_CHUNK):
        for hi, h in enumerate((0, 16)):
            out_v[jj * BAGS_PER_CHUNK + bag, pl.ds(h, 16)] = accs[bag * 2 + hi]


def _bag_sum_body(idx_hbm, weight_hbm, out_hbm, idx_v, out_v, *ring):
    bufs = ring[:NBUF]
    sems = ring[NBUF:]
    wtab = weight_hbm
    wid = lax.axis_index("s") * NUM_CORES + lax.axis_index("c")
    base = wid * BAGS_PER_WORKER

    pltpu.sync_copy(idx_hbm.at[pl.ds(wid * CHUNKS_PER_WORKER,
                                     CHUNKS_PER_WORKER)], idx_v)

    def start(jj, b):
        pltpu.async_copy(wtab.at[idx_v.at[jj]], bufs[b], sems[b])

    def wait(b):
        pltpu.make_async_copy(wtab.at[pl.ds(0, ROWS_PER_CHUNK)], bufs[b],
                              sems[b]).wait()

    for b in range(NBUF):
        start(b, b)

    @pl.loop(0, CHUNKS_PER_WORKER, step=NBUF)
    def _(j):
        for b in range(NBUF):
            jj = j + b
            wait(b)
            _accumulate_bags(out_v, bufs[b], jj)
            nxt = jj + NBUF

            @pl.when(nxt < CHUNKS_PER_WORKER)
            def _():
                start(nxt, b)

    pltpu.sync_copy(out_v, out_hbm.at[pl.ds(base, BAGS_PER_WORKER)])


TR_BLOCK = 8192
SLAB_BLOCKS = 31
SLAB = TR_BLOCK * SLAB_BLOCKS
TAB_ROWS = 4 * SLAB


def _transpose_block(w0, w1, w2, w3, out_ref):
    w4 = jnp.concatenate([w0[...], w1[...], w2[...], w3[...]], axis=0)
    eye = jnp.eye(4 * EMB_DIM, dtype=jnp.float32)
    out_ref[...] = jax.lax.dot_general(
        w4, eye, (((0,), (0,)), ((), ())),
        precision=jax.lax.Precision.HIGHEST,
        preferred_element_type=jnp.float32)


def _to_row_major(wt):
    last_block = VOCAB_ROWS // TR_BLOCK
    specs = [
        pl.BlockSpec((EMB_DIM, TR_BLOCK),
                     lambda i, q=q: (0, jnp.minimum(q * SLAB_BLOCKS + i,
                                                    last_block)))
        for q in range(4)
    ]
    w128 = pl.pallas_call(
        _transpose_block,
        grid=(SLAB_BLOCKS,),
        in_specs=specs,
        out_specs=pl.BlockSpec((TR_BLOCK, 4 * EMB_DIM), lambda i: (i, 0)),
        out_shape=jax.ShapeDtypeStruct((SLAB, 4 * EMB_DIM), jnp.float32),
    )(wt, wt, wt, wt)
    return w128.reshape(TAB_ROWS, EMB_DIM)


def kernel(input, weight):
    idx32 = input.astype(jnp.int32)
    perm = 4 * (idx32 % SLAB) + idx32 // SLAB
    idx = perm.T.reshape(NUM_BAGS // BAGS_PER_CHUNK, ROWS_PER_CHUNK)
    mesh = plsc.VectorSubcoreMesh(core_axis_name="c", subcore_axis_name="s")
    run = pl.kernel(
        _bag_sum_body,
        out_type=jax.ShapeDtypeStruct((NUM_BAGS, EMB_DIM), jnp.float32),
        mesh=mesh,
        compiler_params=pltpu.CompilerParams(use_tc_tiling_on_sc=False),
        scratch_types=(
            [pltpu.VMEM((CHUNKS_PER_WORKER, ROWS_PER_CHUNK), jnp.int32),
             pltpu.VMEM((BAGS_PER_WORKER, EMB_DIM), jnp.float32)]
            + [pltpu.VMEM((ROWS_PER_CHUNK, EMB_DIM), jnp.float32)] * NBUF
            + [pltpu.SemaphoreType.DMA] * NBUF
        ),
    )
    return run(idx, _to_row_major(jnp.swapaxes(weight, 0, 1)))
